# baseline (speedup 1.0000x reference)
import jax
import jax.numpy as jnp
from jax import lax
from jax.experimental import pallas as pl
from jax.experimental.pallas import tpu as pltpu

B = 8
N_OBJ = 50
NUM_CLS = 10
H = 180
W = 180
OUT_SIZE_FACTOR = 4
MIN_RADIUS = 2
GAUSSIAN_OVERLAP = 0.1
EPS = 1e-4
WIN = 32
HP = 184
NF = 14
NI = 3


def _params_body(gtt_ref, fp_ref, ip_ref):
    def g(j):
        return gtt_ref[j]

    cx = g(0) * (1.0 / OUT_SIZE_FACTOR)
    cy = g(1) * (1.0 / OUT_SIZE_FACTOR)
    bw_ = g(3) * (1.0 / OUT_SIZE_FACTOR)
    bl = g(4) * (1.0 / OUT_SIZE_FACTOR)

    h_, w_ = bl, bw_
    ov = GAUSSIAN_OVERLAP
    b1 = h_ + w_
    c1 = w_ * h_ * (1.0 - ov) / (1.0 + ov)
    sq1 = jnp.sqrt(jnp.maximum(b1 * b1 - 4.0 * c1, 0.0))
    r1 = (b1 + sq1) / 2.0
    b2 = 2.0 * (h_ + w_)
    c2 = (1.0 - ov) * w_ * h_
    sq2 = jnp.sqrt(jnp.maximum(b2 * b2 - 16.0 * c2, 0.0))
    r2 = (b2 + sq2) / 2.0
    a3 = 4.0 * ov
    b3 = -2.0 * ov * (h_ + w_)
    c3 = (ov - 1.0) * w_ * h_
    sq3 = jnp.sqrt(jnp.maximum(b3 * b3 - 4.0 * a3 * c3, 0.0))
    r3 = (b3 + sq3) / (2.0 * a3)
    rad = jnp.minimum(jnp.minimum(r1, r2), r3)
    r = jnp.maximum(float(MIN_RADIUS), jnp.floor(rad))

    icx = jnp.floor(cx)
    icy = jnp.floor(cy)
    valid = (bw_ > 0) & (bl > 0) & (icx >= 0) & (icx < W) & (icy >= 0) & (icy < H)
    sigma = (2.0 * r + 1.0) / 6.0
    inv2s2 = 1.0 / (2.0 * sigma * sigma)

    icyi = icy.astype(jnp.int32)
    srow = jnp.clip(((icyi - 11) // 8) * 8, 0, HP - WIN)

    fp_ref[:, 0, :] = cx
    fp_ref[:, 1, :] = cy
    fp_ref[:, 2, :] = icx
    fp_ref[:, 3, :] = icy
    fp_ref[:, 4, :] = r
    fp_ref[:, 5, :] = inv2s2
    fp_ref[:, 6, :] = g(2)
    fp_ref[:, 7, :] = jnp.log(jnp.maximum(g(3), 1e-6))
    fp_ref[:, 8, :] = jnp.log(jnp.maximum(g(4), 1e-6))
    fp_ref[:, 9, :] = jnp.log(jnp.maximum(g(5), 1e-6))
    fp_ref[:, 10, :] = jnp.sin(g(6))
    fp_ref[:, 11, :] = jnp.cos(g(6))
    fp_ref[:, 12, :] = g(7)
    fp_ref[:, 13, :] = g(8)
    ip_ref[:, 0, :] = g(9).astype(jnp.int32)
    ip_ref[:, 1, :] = srow
    ip_ref[:, 2, :] = valid.astype(jnp.int32)


def _main_body(hm_in, fp, ip, wqf, cqf, reg_i, hei_i, dim_i, rot_i, vel_i,
               logits_o, hms_o, bt_o, bw_o, pred_o, avg_o):
    b = pl.program_id(0)

    x = hm_in[0]
    logits_o[0] = jnp.clip(1.0 / (1.0 + jnp.exp(-x)), EPS, 1.0 - EPS)

    planes = [reg_i[0, 0], reg_i[0, 1], hei_i[0, 0],
              dim_i[0, 0], dim_i[0, 1], dim_i[0, 2],
              rot_i[0, 0], rot_i[0, 1], vel_i[0, 0], vel_i[0, 1]]
    kq = lax.broadcasted_iota(jnp.int32, (W, W * NUM_CLS), 0)
    lq = lax.broadcasted_iota(jnp.int32, (W, W * NUM_CLS), 1)
    acc = jnp.zeros((H, W * NUM_CLS), jnp.float32)
    for c in range(NUM_CLS):
        p_c = (lq == NUM_CLS * kq + c).astype(jnp.float32)
        acc = acc + jax.lax.dot(planes[c], p_c,
                                precision=jax.lax.Precision.HIGHEST,
                                preferred_element_type=jnp.float32)
    pred_o[0] = acc

    hms_o[0] = jnp.zeros((NUM_CLS, HP, W), jnp.float32)
    bt_o[0] = jnp.zeros((HP, W * NUM_CLS), jnp.float32)
    bw_o[0] = jnp.zeros((HP, W * NUM_CLS), jnp.float32)

    wq = wqf[:, :]
    cq = cqf[:, :]

    def obj(n, carry):
        valid = ip[0, 2, n] == 1

        @pl.when(valid)
        def _():
            c = ip[0, 0, n]
            s = pl.multiple_of(ip[0, 1, n], 8)
            cx = fp[0, 0, n]
            cy = fp[0, 1, n]
            icx = fp[0, 2, n]
            icy = fp[0, 3, n]
            r = fp[0, 4, n]
            k2 = fp[0, 5, n]

            hv = s.astype(jnp.float32) + lax.broadcasted_iota(jnp.int32, (WIN, 1), 0).astype(jnp.float32)
            dy = hv - icy
            rowm = jnp.abs(dy) <= r
            gy = jnp.exp(-(dy * dy) * k2)

            xw = lax.broadcasted_iota(jnp.int32, (1, W), 1).astype(jnp.float32)
            dxn = xw - icx
            colmn = jnp.abs(dxn) <= r
            gn = gy * jnp.exp(-(dxn * dxn) * k2)
            gn = jnp.where(rowm & colmn, gn, 0.0)
            cur = hms_o[0, c, pl.ds(s, WIN), :]
            hms_o[0, c, pl.ds(s, WIN), :] = jnp.maximum(cur, gn)

            dxe = wq - icx
            colme = jnp.abs(dxe) <= r
            ge = gy * jnp.exp(-(dxe * dxe) * k2)
            m2 = rowm & colme
            bwcur = bw_o[0, pl.ds(s, WIN), :]
            bw_o[0, pl.ds(s, WIN), :] = jnp.maximum(bwcur, jnp.where(m2, ge, 0.0))

            crow = jnp.where(cq == 2.0, fp[0, 6, n],
                   jnp.where(cq == 3.0, fp[0, 7, n],
                   jnp.where(cq == 4.0, fp[0, 8, n],
                   jnp.where(cq == 5.0, fp[0, 9, n],
                   jnp.where(cq == 6.0, fp[0, 10, n],
                   jnp.where(cq == 7.0, fp[0, 11, n],
                   jnp.where(cq == 8.0, fp[0, 12, n], fp[0, 13, n])))))))
            val = jnp.where(cq == 0.0, cx - wq, jnp.where(cq == 1.0, cy - hv, crow))
            btcur = bt_o[0, pl.ds(s, WIN), :]
            bt_o[0, pl.ds(s, WIN), :] = jnp.where(m2, val, btcur)

        return carry

    lax.fori_loop(0, N_OBJ, obj, 0)

    bwn = jnp.max(hms_o[0, :, 0:H, :], axis=0)
    ps = jnp.sum(bwn)
    prev = jnp.where(b == 0, 0.0, avg_o[0, 0])
    tot = prev + ps
    avg_o[0, 0] = jnp.where(b == B - 1, jnp.maximum(tot, 1.0), tot)


def kernel(gt_bboxes_3d, heatmap, reg, height, dim, rot, vel):
    gtt = jnp.transpose(gt_bboxes_3d, (2, 0, 1))

    fp, ip = pl.pallas_call(
        _params_body,
        out_shape=(
            jax.ShapeDtypeStruct((B, NF, N_OBJ), jnp.float32),
            jax.ShapeDtypeStruct((B, NI, N_OBJ), jnp.int32),
        ),
    )(gtt)

    wqf = (jnp.arange(W * NUM_CLS, dtype=jnp.int32) // NUM_CLS).astype(jnp.float32)[None, :]
    cqf = (jnp.arange(W * NUM_CLS, dtype=jnp.int32) % NUM_CLS).astype(jnp.float32)[None, :]

    grid = (B,)
    logits, hms, bt, bw, pred_e, avg = pl.pallas_call(
        _main_body,
        grid=grid,
        in_specs=[
            pl.BlockSpec((1, NUM_CLS, H, W), lambda b: (b, 0, 0, 0)),
            pl.BlockSpec((1, NF, N_OBJ), lambda b: (b, 0, 0), memory_space=pltpu.SMEM),
            pl.BlockSpec((1, NI, N_OBJ), lambda b: (b, 0, 0), memory_space=pltpu.SMEM),
            pl.BlockSpec((1, W * NUM_CLS), lambda b: (0, 0)),
            pl.BlockSpec((1, W * NUM_CLS), lambda b: (0, 0)),
            pl.BlockSpec((1, 2, H, W), lambda b: (b, 0, 0, 0)),
            pl.BlockSpec((1, 1, H, W), lambda b: (b, 0, 0, 0)),
            pl.BlockSpec((1, 3, H, W), lambda b: (b, 0, 0, 0)),
            pl.BlockSpec((1, 2, H, W), lambda b: (b, 0, 0, 0)),
            pl.BlockSpec((1, 2, H, W), lambda b: (b, 0, 0, 0)),
        ],
        out_specs=(
            pl.BlockSpec((1, NUM_CLS, H, W), lambda b: (b, 0, 0, 0)),
            pl.BlockSpec((1, NUM_CLS, HP, W), lambda b: (b, 0, 0, 0)),
            pl.BlockSpec((1, HP, W * NUM_CLS), lambda b: (b, 0, 0)),
            pl.BlockSpec((1, HP, W * NUM_CLS), lambda b: (b, 0, 0)),
            pl.BlockSpec((1, H, W * NUM_CLS), lambda b: (b, 0, 0)),
            pl.BlockSpec((1, 1), lambda b: (0, 0), memory_space=pltpu.SMEM),
        ),
        out_shape=(
            jax.ShapeDtypeStruct((B, NUM_CLS, H, W), jnp.float32),
            jax.ShapeDtypeStruct((B, NUM_CLS, H, W), jnp.float32),
            jax.ShapeDtypeStruct((B, H, W * NUM_CLS), jnp.float32),
            jax.ShapeDtypeStruct((B, H, W * NUM_CLS), jnp.float32),
            jax.ShapeDtypeStruct((B, H, W * NUM_CLS), jnp.float32),
            jax.ShapeDtypeStruct((1, 1), jnp.float32),
        ),
    )(heatmap, fp, ip, wqf, cqf, reg, height, dim, rot, vel)

    bbox_pred = pred_e.reshape(B, H, W, NUM_CLS)
    bbox_targets = bt.reshape(B, H, W, NUM_CLS)
    bbox_weight = bw.reshape(B, H, W, NUM_CLS)
    avg_factor = avg[0, 0]
    return logits, hms, bbox_pred, bbox_targets, bbox_weight, avg_factor


# speedup vs baseline: 1.0973x; 1.0973x over previous
import jax
import jax.numpy as jnp
from jax import lax
from jax.experimental import pallas as pl
from jax.experimental.pallas import tpu as pltpu

B = 8
N_OBJ = 50
NUM_CLS = 10
H = 180
W = 180
OUT_SIZE_FACTOR = 4
MIN_RADIUS = 2
GAUSSIAN_OVERLAP = 0.1
EPS = 1e-4
WIN = 32
HP = 184
NF = 14
NI = 4
WE = W * NUM_CLS
WEP = 1920
SLAB = 384


def _params_body(gtt_ref, fp_ref, ip_ref):
    def g(j):
        return gtt_ref[j]

    cx = g(0) * (1.0 / OUT_SIZE_FACTOR)
    cy = g(1) * (1.0 / OUT_SIZE_FACTOR)
    bw_ = g(3) * (1.0 / OUT_SIZE_FACTOR)
    bl = g(4) * (1.0 / OUT_SIZE_FACTOR)

    h_, w_ = bl, bw_
    ov = GAUSSIAN_OVERLAP
    b1 = h_ + w_
    c1 = w_ * h_ * (1.0 - ov) / (1.0 + ov)
    sq1 = jnp.sqrt(jnp.maximum(b1 * b1 - 4.0 * c1, 0.0))
    r1 = (b1 + sq1) / 2.0
    b2 = 2.0 * (h_ + w_)
    c2 = (1.0 - ov) * w_ * h_
    sq2 = jnp.sqrt(jnp.maximum(b2 * b2 - 16.0 * c2, 0.0))
    r2 = (b2 + sq2) / 2.0
    a3 = 4.0 * ov
    b3 = -2.0 * ov * (h_ + w_)
    c3 = (ov - 1.0) * w_ * h_
    sq3 = jnp.sqrt(jnp.maximum(b3 * b3 - 4.0 * a3 * c3, 0.0))
    r3 = (b3 + sq3) / (2.0 * a3)
    rad = jnp.minimum(jnp.minimum(r1, r2), r3)
    r = jnp.maximum(float(MIN_RADIUS), jnp.floor(rad))

    icx = jnp.floor(cx)
    icy = jnp.floor(cy)
    valid = (bw_ > 0) & (bl > 0) & (icx >= 0) & (icx < W) & (icy >= 0) & (icy < H)
    sigma = (2.0 * r + 1.0) / 6.0
    inv2s2 = 1.0 / (2.0 * sigma * sigma)

    icyi = icy.astype(jnp.int32)
    srow = jnp.clip(((icyi - 11) // 8) * 8, 0, HP - WIN)
    icxi = icx.astype(jnp.int32)
    w0 = NUM_CLS * (icxi - 11)
    slane = jnp.clip((w0 // 128) * 128, 0, WEP - SLAB)

    fp_ref[:, 0, :] = cx
    fp_ref[:, 1, :] = cy
    fp_ref[:, 2, :] = icx
    fp_ref[:, 3, :] = icy
    fp_ref[:, 4, :] = r
    fp_ref[:, 5, :] = inv2s2
    fp_ref[:, 6, :] = g(2)
    fp_ref[:, 7, :] = jnp.log(jnp.maximum(g(3), 1e-6))
    fp_ref[:, 8, :] = jnp.log(jnp.maximum(g(4), 1e-6))
    fp_ref[:, 9, :] = jnp.log(jnp.maximum(g(5), 1e-6))
    fp_ref[:, 10, :] = jnp.sin(g(6))
    fp_ref[:, 11, :] = jnp.cos(g(6))
    fp_ref[:, 12, :] = g(7)
    fp_ref[:, 13, :] = g(8)
    ip_ref[:, 0, :] = g(9).astype(jnp.int32)
    ip_ref[:, 1, :] = srow
    ip_ref[:, 2, :] = valid.astype(jnp.int32)
    ip_ref[:, 3, :] = slane


def _main_body(hm_in, fp, ip, wqf, cqf, reg_i, hei_i, dim_i, rot_i, vel_i,
               logits_o, hms_o, bt_o, bw_o, pred_o, avg_o):
    b = pl.program_id(0)

    x = hm_in[0]
    logits_o[0] = jnp.clip(1.0 / (1.0 + jnp.exp(-x)), EPS, 1.0 - EPS)

    planes = [reg_i[0, 0], reg_i[0, 1], hei_i[0, 0],
              dim_i[0, 0], dim_i[0, 1], dim_i[0, 2],
              rot_i[0, 0], rot_i[0, 1], vel_i[0, 0], vel_i[0, 1]]
    kq = lax.broadcasted_iota(jnp.int32, (W, W * NUM_CLS), 0)
    lq = lax.broadcasted_iota(jnp.int32, (W, W * NUM_CLS), 1)
    acc = jnp.zeros((H, W * NUM_CLS), jnp.float32)
    for c in range(NUM_CLS):
        p_c = (lq == NUM_CLS * kq + c).astype(jnp.float32)
        acc = acc + jax.lax.dot(planes[c], p_c,
                                precision=jax.lax.Precision.HIGHEST,
                                preferred_element_type=jnp.float32)
    pred_o[0] = acc

    hms_o[0] = jnp.zeros((NUM_CLS, HP, W), jnp.float32)
    bt_o[0] = jnp.zeros((HP, WEP), jnp.float32)
    bw_o[0] = jnp.zeros((HP, WEP), jnp.float32)

    def obj(n, carry):
        valid = ip[0, 2, n] == 1

        @pl.when(valid)
        def _():
            c = ip[0, 0, n]
            s = pl.multiple_of(ip[0, 1, n], 8)
            cx = fp[0, 0, n]
            cy = fp[0, 1, n]
            icx = fp[0, 2, n]
            icy = fp[0, 3, n]
            r = fp[0, 4, n]
            k2 = fp[0, 5, n]

            hv = s.astype(jnp.float32) + lax.broadcasted_iota(jnp.int32, (WIN, 1), 0).astype(jnp.float32)
            dy = hv - icy
            rowm = jnp.abs(dy) <= r
            gy = jnp.exp(-(dy * dy) * k2)

            xw = lax.broadcasted_iota(jnp.int32, (1, W), 1).astype(jnp.float32)
            dxn = xw - icx
            colmn = jnp.abs(dxn) <= r
            gn = gy * jnp.exp(-(dxn * dxn) * k2)
            gn = jnp.where(rowm & colmn, gn, 0.0)
            cur = hms_o[0, c, pl.ds(s, WIN), :]
            hms_o[0, c, pl.ds(s, WIN), :] = jnp.maximum(cur, gn)

            ls = pl.multiple_of(ip[0, 3, n], 128)
            wq = wqf[:, pl.ds(ls, SLAB)]
            cq = cqf[:, pl.ds(ls, SLAB)]
            dxe = wq - icx
            colme = jnp.abs(dxe) <= r
            ge = gy * jnp.exp(-(dxe * dxe) * k2)
            m2 = rowm & colme
            bwcur = bw_o[0, pl.ds(s, WIN), pl.ds(ls, SLAB)]
            bw_o[0, pl.ds(s, WIN), pl.ds(ls, SLAB)] = jnp.maximum(
                bwcur, jnp.where(m2, ge, 0.0))

            crow = jnp.where(cq == 2.0, fp[0, 6, n],
                   jnp.where(cq == 3.0, fp[0, 7, n],
                   jnp.where(cq == 4.0, fp[0, 8, n],
                   jnp.where(cq == 5.0, fp[0, 9, n],
                   jnp.where(cq == 6.0, fp[0, 10, n],
                   jnp.where(cq == 7.0, fp[0, 11, n],
                   jnp.where(cq == 8.0, fp[0, 12, n], fp[0, 13, n])))))))
            val = jnp.where(cq == 0.0, cx - wq, jnp.where(cq == 1.0, cy - hv, crow))
            btcur = bt_o[0, pl.ds(s, WIN), pl.ds(ls, SLAB)]
            bt_o[0, pl.ds(s, WIN), pl.ds(ls, SLAB)] = jnp.where(m2, val, btcur)

        return carry

    lax.fori_loop(0, N_OBJ, obj, 0)

    bwn = jnp.max(hms_o[0, :, 0:H, :], axis=0)
    ps = jnp.sum(bwn)
    prev = jnp.where(b == 0, 0.0, avg_o[0, 0])
    tot = prev + ps
    avg_o[0, 0] = jnp.where(b == B - 1, jnp.maximum(tot, 1.0), tot)


def kernel(gt_bboxes_3d, heatmap, reg, height, dim, rot, vel):
    gtt = jnp.transpose(gt_bboxes_3d, (2, 0, 1))

    fp, ip = pl.pallas_call(
        _params_body,
        out_shape=(
            jax.ShapeDtypeStruct((B, NF, N_OBJ), jnp.float32),
            jax.ShapeDtypeStruct((B, NI, N_OBJ), jnp.int32),
        ),
    )(gtt)

    wqf = (jnp.arange(WEP, dtype=jnp.int32) // NUM_CLS).astype(jnp.float32)[None, :]
    cqf = (jnp.arange(WEP, dtype=jnp.int32) % NUM_CLS).astype(jnp.float32)[None, :]

    grid = (B,)
    logits, hms, bt, bw, pred_e, avg = pl.pallas_call(
        _main_body,
        grid=grid,
        in_specs=[
            pl.BlockSpec((1, NUM_CLS, H, W), lambda b: (b, 0, 0, 0)),
            pl.BlockSpec((1, NF, N_OBJ), lambda b: (b, 0, 0), memory_space=pltpu.SMEM),
            pl.BlockSpec((1, NI, N_OBJ), lambda b: (b, 0, 0), memory_space=pltpu.SMEM),
            pl.BlockSpec((1, WEP), lambda b: (0, 0)),
            pl.BlockSpec((1, WEP), lambda b: (0, 0)),
            pl.BlockSpec((1, 2, H, W), lambda b: (b, 0, 0, 0)),
            pl.BlockSpec((1, 1, H, W), lambda b: (b, 0, 0, 0)),
            pl.BlockSpec((1, 3, H, W), lambda b: (b, 0, 0, 0)),
            pl.BlockSpec((1, 2, H, W), lambda b: (b, 0, 0, 0)),
            pl.BlockSpec((1, 2, H, W), lambda b: (b, 0, 0, 0)),
        ],
        out_specs=(
            pl.BlockSpec((1, NUM_CLS, H, W), lambda b: (b, 0, 0, 0)),
            pl.BlockSpec((1, NUM_CLS, HP, W), lambda b: (b, 0, 0, 0)),
            pl.BlockSpec((1, HP, WEP), lambda b: (b, 0, 0)),
            pl.BlockSpec((1, HP, WEP), lambda b: (b, 0, 0)),
            pl.BlockSpec((1, H, W * NUM_CLS), lambda b: (b, 0, 0)),
            pl.BlockSpec((1, 1), lambda b: (0, 0), memory_space=pltpu.SMEM),
        ),
        out_shape=(
            jax.ShapeDtypeStruct((B, NUM_CLS, H, W), jnp.float32),
            jax.ShapeDtypeStruct((B, NUM_CLS, H, W), jnp.float32),
            jax.ShapeDtypeStruct((B, H, W * NUM_CLS), jnp.float32),
            jax.ShapeDtypeStruct((B, H, W * NUM_CLS), jnp.float32),
            jax.ShapeDtypeStruct((B, H, W * NUM_CLS), jnp.float32),
            jax.ShapeDtypeStruct((1, 1), jnp.float32),
        ),
    )(heatmap, fp, ip, wqf, cqf, reg, height, dim, rot, vel)

    bbox_pred = pred_e.reshape(B, H, W, NUM_CLS)
    bbox_targets = bt.reshape(B, H, W, NUM_CLS)
    bbox_weight = bw.reshape(B, H, W, NUM_CLS)
    avg_factor = avg[0, 0]
    return logits, hms, bbox_pred, bbox_targets, bbox_weight, avg_factor


# speedup vs baseline: 2.0290x; 1.8490x over previous
import jax
import jax.numpy as jnp
from jax import lax
from jax.experimental import pallas as pl
from jax.experimental.pallas import tpu as pltpu

B = 8
N_OBJ = 50
NUM_CLS = 10
H = 180
W = 180
OUT_SIZE_FACTOR = 4
MIN_RADIUS = 2
GAUSSIAN_OVERLAP = 0.1
EPS = 1e-4
WIN = 32
HP = 184
NF = 14
NI = 4
WE = W * NUM_CLS
WEP = 1920
SLAB = 384


def _params_body(gtt_ref, fp_ref, ip_ref):
    def g(j):
        return gtt_ref[j]

    cx = g(0) * (1.0 / OUT_SIZE_FACTOR)
    cy = g(1) * (1.0 / OUT_SIZE_FACTOR)
    bw_ = g(3) * (1.0 / OUT_SIZE_FACTOR)
    bl = g(4) * (1.0 / OUT_SIZE_FACTOR)

    h_, w_ = bl, bw_
    ov = GAUSSIAN_OVERLAP
    b1 = h_ + w_
    c1 = w_ * h_ * (1.0 - ov) / (1.0 + ov)
    sq1 = jnp.sqrt(jnp.maximum(b1 * b1 - 4.0 * c1, 0.0))
    r1 = (b1 + sq1) / 2.0
    b2 = 2.0 * (h_ + w_)
    c2 = (1.0 - ov) * w_ * h_
    sq2 = jnp.sqrt(jnp.maximum(b2 * b2 - 16.0 * c2, 0.0))
    r2 = (b2 + sq2) / 2.0
    a3 = 4.0 * ov
    b3 = -2.0 * ov * (h_ + w_)
    c3 = (ov - 1.0) * w_ * h_
    sq3 = jnp.sqrt(jnp.maximum(b3 * b3 - 4.0 * a3 * c3, 0.0))
    r3 = (b3 + sq3) / (2.0 * a3)
    rad = jnp.minimum(jnp.minimum(r1, r2), r3)
    r = jnp.maximum(float(MIN_RADIUS), jnp.floor(rad))

    icx = jnp.floor(cx)
    icy = jnp.floor(cy)
    valid = (bw_ > 0) & (bl > 0) & (icx >= 0) & (icx < W) & (icy >= 0) & (icy < H)
    sigma = (2.0 * r + 1.0) / 6.0
    inv2s2 = 1.0 / (2.0 * sigma * sigma)

    icyi = icy.astype(jnp.int32)
    srow = jnp.clip(((icyi - 11) // 8) * 8, 0, HP - WIN)
    icxi = icx.astype(jnp.int32)
    w0 = NUM_CLS * (icxi - 11)
    slane = jnp.clip((w0 // 128) * 128, 0, WEP - SLAB)

    fp_ref[:, 0, :] = cx
    fp_ref[:, 1, :] = cy
    fp_ref[:, 2, :] = icx
    fp_ref[:, 3, :] = icy
    fp_ref[:, 4, :] = r
    fp_ref[:, 5, :] = inv2s2
    fp_ref[:, 6, :] = g(2)
    fp_ref[:, 7, :] = jnp.log(jnp.maximum(g(3), 1e-6))
    fp_ref[:, 8, :] = jnp.log(jnp.maximum(g(4), 1e-6))
    fp_ref[:, 9, :] = jnp.log(jnp.maximum(g(5), 1e-6))
    fp_ref[:, 10, :] = jnp.sin(g(6))
    fp_ref[:, 11, :] = jnp.cos(g(6))
    fp_ref[:, 12, :] = g(7)
    fp_ref[:, 13, :] = g(8)
    ip_ref[:, 0, :] = g(9).astype(jnp.int32)
    ip_ref[:, 1, :] = srow
    ip_ref[:, 2, :] = valid.astype(jnp.int32)
    ip_ref[:, 3, :] = slane


def _main_body(hm_in, fp, ip, wqf, cqf, logits_o, hms_o, bt_o, bw_o, avg_o):
    b = pl.program_id(0)

    x = hm_in[0]
    logits_o[0] = jnp.clip(1.0 / (1.0 + jnp.exp(-x)), EPS, 1.0 - EPS)

    hms_o[0] = jnp.zeros((NUM_CLS, HP, W), jnp.float32)
    bt_o[0] = jnp.zeros((HP, WEP), jnp.float32)
    bw_o[0] = jnp.zeros((HP, WEP), jnp.float32)

    def obj(n, carry):
        valid = ip[0, 2, n] == 1

        @pl.when(valid)
        def _():
            c = ip[0, 0, n]
            s = pl.multiple_of(ip[0, 1, n], 8)
            cx = fp[0, 0, n]
            cy = fp[0, 1, n]
            icx = fp[0, 2, n]
            icy = fp[0, 3, n]
            r = fp[0, 4, n]
            k2 = fp[0, 5, n]

            hv = s.astype(jnp.float32) + lax.broadcasted_iota(jnp.int32, (WIN, 1), 0).astype(jnp.float32)
            dy = hv - icy
            rowm = jnp.abs(dy) <= r
            gy = jnp.exp(-(dy * dy) * k2)

            xw = lax.broadcasted_iota(jnp.int32, (1, W), 1).astype(jnp.float32)
            dxn = xw - icx
            colmn = jnp.abs(dxn) <= r
            gn = gy * jnp.exp(-(dxn * dxn) * k2)
            gn = jnp.where(rowm & colmn, gn, 0.0)
            cur = hms_o[0, c, pl.ds(s, WIN), :]
            hms_o[0, c, pl.ds(s, WIN), :] = jnp.maximum(cur, gn)

            ls = pl.multiple_of(ip[0, 3, n], 128)
            wq = wqf[:, pl.ds(ls, SLAB)]
            cq = cqf[:, pl.ds(ls, SLAB)]
            dxe = wq - icx
            colme = jnp.abs(dxe) <= r
            ge = gy * jnp.exp(-(dxe * dxe) * k2)
            m2 = rowm & colme
            bwcur = bw_o[0, pl.ds(s, WIN), pl.ds(ls, SLAB)]
            bw_o[0, pl.ds(s, WIN), pl.ds(ls, SLAB)] = jnp.maximum(
                bwcur, jnp.where(m2, ge, 0.0))

            crow = jnp.where(cq == 2.0, fp[0, 6, n],
                   jnp.where(cq == 3.0, fp[0, 7, n],
                   jnp.where(cq == 4.0, fp[0, 8, n],
                   jnp.where(cq == 5.0, fp[0, 9, n],
                   jnp.where(cq == 6.0, fp[0, 10, n],
                   jnp.where(cq == 7.0, fp[0, 11, n],
                   jnp.where(cq == 8.0, fp[0, 12, n], fp[0, 13, n])))))))
            val = jnp.where(cq == 0.0, cx - wq, jnp.where(cq == 1.0, cy - hv, crow))
            btcur = bt_o[0, pl.ds(s, WIN), pl.ds(ls, SLAB)]
            bt_o[0, pl.ds(s, WIN), pl.ds(ls, SLAB)] = jnp.where(m2, val, btcur)

        return carry

    lax.fori_loop(0, N_OBJ, obj, 0)

    bwn = jnp.max(hms_o[0, :, 0:H, :], axis=0)
    ps = jnp.sum(bwn)
    prev = jnp.where(b == 0, 0.0, avg_o[0, 0])
    tot = prev + ps
    avg_o[0, 0] = jnp.where(b == B - 1, jnp.maximum(tot, 1.0), tot)


def kernel(gt_bboxes_3d, heatmap, reg, height, dim, rot, vel):
    gtt = jnp.transpose(gt_bboxes_3d, (2, 0, 1))

    fp, ip = pl.pallas_call(
        _params_body,
        out_shape=(
            jax.ShapeDtypeStruct((B, NF, N_OBJ), jnp.float32),
            jax.ShapeDtypeStruct((B, NI, N_OBJ), jnp.int32),
        ),
    )(gtt)

    wqf = (jnp.arange(WEP, dtype=jnp.int32) // NUM_CLS).astype(jnp.float32)[None, :]
    cqf = (jnp.arange(WEP, dtype=jnp.int32) % NUM_CLS).astype(jnp.float32)[None, :]

    grid = (B,)
    logits, hms, bt, bw, avg = pl.pallas_call(
        _main_body,
        grid=grid,
        in_specs=[
            pl.BlockSpec((1, NUM_CLS, H, W), lambda b: (b, 0, 0, 0)),
            pl.BlockSpec((1, NF, N_OBJ), lambda b: (b, 0, 0), memory_space=pltpu.SMEM),
            pl.BlockSpec((1, NI, N_OBJ), lambda b: (b, 0, 0), memory_space=pltpu.SMEM),
            pl.BlockSpec((1, WEP), lambda b: (0, 0)),
            pl.BlockSpec((1, WEP), lambda b: (0, 0)),
        ],
        out_specs=(
            pl.BlockSpec((1, NUM_CLS, H, W), lambda b: (b, 0, 0, 0)),
            pl.BlockSpec((1, NUM_CLS, HP, W), lambda b: (b, 0, 0, 0)),
            pl.BlockSpec((1, HP, WEP), lambda b: (b, 0, 0)),
            pl.BlockSpec((1, HP, WEP), lambda b: (b, 0, 0)),
            pl.BlockSpec((1, 1), lambda b: (0, 0), memory_space=pltpu.SMEM),
        ),
        out_shape=(
            jax.ShapeDtypeStruct((B, NUM_CLS, H, W), jnp.float32),
            jax.ShapeDtypeStruct((B, NUM_CLS, H, W), jnp.float32),
            jax.ShapeDtypeStruct((B, H, W * NUM_CLS), jnp.float32),
            jax.ShapeDtypeStruct((B, H, W * NUM_CLS), jnp.float32),
            jax.ShapeDtypeStruct((1, 1), jnp.float32),
        ),
    )(heatmap, fp, ip, wqf, cqf)

    bbox_pred = jnp.transpose(
        jnp.concatenate([reg, height, dim, rot, vel], axis=1), (0, 2, 3, 1)
    )
    bbox_targets = bt.reshape(B, H, W, NUM_CLS)
    bbox_weight = bw.reshape(B, H, W, NUM_CLS)
    avg_factor = avg[0, 0]
    return logits, hms, bbox_pred, bbox_targets, bbox_weight, avg_factor
